# baseline (speedup 1.0000x reference)
import functools

import jax
import jax.numpy as jnp
from jax import lax
from jax.experimental import pallas as pl
from jax.experimental.pallas import tpu as pltpu
from jax.experimental.pallas import tpu_sc as plsc

N_HID = 128
LANES = 16
NC = 2
NS = 16
NW = NC * NS
CHUNK = 128


def _project_table(emb, W, b2):
    M = emb.shape[0]
    BM = 2048
    grid = (pl.cdiv(M, BM),)
    H = N_HID // 2

    def body(e_ref, w_ref, b_ref, o_ref):
        acc = jax.lax.dot_general(
            e_ref[...], w_ref[...],
            dimension_numbers=(((1,), (1,)), ((), ())),
            preferred_element_type=jnp.float32,
        )
        y = acc + b_ref[0, :][None, :]
        lo = jax.lax.bitcast_convert_type(
            y[:, :H].astype(jnp.bfloat16), jnp.uint16).astype(jnp.uint32)
        hi = jax.lax.bitcast_convert_type(
            y[:, H:].astype(jnp.bfloat16), jnp.uint16).astype(jnp.uint32)
        o_ref[...] = (lo | (hi << 16)).astype(jnp.int32)

    return pl.pallas_call(
        body,
        grid=grid,
        in_specs=[
            pl.BlockSpec((BM, N_HID), lambda i: (i, 0)),
            pl.BlockSpec((N_HID, N_HID), lambda i: (0, 0)),
            pl.BlockSpec((8, N_HID), lambda i: (0, 0)),
        ],
        out_specs=pl.BlockSpec((BM, H), lambda i: (i, 0)),
        out_shape=jax.ShapeDtypeStruct((M, H), jnp.int32),
    )(emb, W, b2)


def _gather_add(P, t, x):
    N = t.shape[0]
    assert N % NW == 0
    rows_w = N // NW
    nfull = rows_w // CHUNK
    rem = rows_w - nfull * CHUNK
    assert rem % 8 == 0 and nfull >= 4 and nfull % 2 == 0

    mesh = plsc.VectorSubcoreMesh(core_axis_name="c", subcore_axis_name="s")

    @functools.partial(
        pl.kernel,
        mesh=mesh,
        out_type=jax.ShapeDtypeStruct((N, N_HID), jnp.float32),
        compiler_params=pltpu.CompilerParams(
            needs_layout_passes=False, use_tc_tiling_on_sc=False),
        scratch_types=[
            pltpu.VMEM((rows_w,), jnp.int32),
            pltpu.VMEM((2, CHUNK, N_HID // 2), jnp.int32),
            pltpu.VMEM((2, CHUNK, N_HID), jnp.float32),
            pltpu.VMEM((2, CHUNK, N_HID), jnp.float32),
            pltpu.SemaphoreType.DMA,
            pltpu.SemaphoreType.DMA,
            pltpu.SemaphoreType.DMA,
            pltpu.SemaphoreType.DMA,
            pltpu.SemaphoreType.DMA,
            pltpu.SemaphoreType.DMA,
        ],
    )
    def k(p_hbm, t_hbm, x_hbm, out_hbm, t_v, rows_v, x_v, o_v,
          g0, g1, xs0, xs1, os0, os1):
        wid = lax.axis_index("s") * NC + lax.axis_index("c")
        wbase = wid * rows_w
        gsem = (g0, g1)
        xsem = (xs0, xs1)
        osem = (os0, os1)

        pltpu.sync_copy(t_hbm.at[pl.ds(wbase, rows_w)], t_v)

        def gather_desc(c, s):
            return pltpu.make_async_copy(
                p_hbm.at[t_v.at[pl.ds(c * CHUNK, CHUNK)]], rows_v.at[s],
                gsem[s])

        def x_desc(c, s):
            return pltpu.make_async_copy(
                x_hbm.at[pl.ds(wbase + c * CHUNK, CHUNK)], x_v.at[s], xsem[s])

        def o_desc(c, s):
            return pltpu.make_async_copy(
                o_v.at[s], out_hbm.at[pl.ds(wbase + c * CHUNK, CHUNK)],
                osem[s])

        def issue(c, s):
            gather_desc(c, s).start()
            x_desc(c, s).start()

        def unpack_add_row(s, r):
            for j in range(N_HID // (2 * LANES)):
                sl = pl.ds(j * LANES, LANES)
                sh = pl.ds(N_HID // 2 + j * LANES, LANES)
                v = rows_v[s, r, sl]
                f_lo = plsc.bitcast(v << jnp.int32(16), jnp.float32)
                f_hi = plsc.bitcast(v & jnp.int32(-65536), jnp.float32)
                o_v[s, r, sl] = x_v[s, r, sl] + f_lo
                o_v[s, r, sh] = x_v[s, r, sh] + f_hi

        def add_chunk(s):
            def add_row(r, _):
                unpack_add_row(s, r)
                return 0
            lax.fori_loop(0, CHUNK, add_row, 0)

        def process(c, s, prefetch, wait_store):
            gather_desc(c, s).wait()
            x_desc(c, s).wait()
            if prefetch:
                issue(c + 1, 1 - s)
            if wait_store:
                o_desc(c - 2, s).wait()
            add_chunk(s)
            o_desc(c, s).start()

        issue(0, 0)
        process(0, 0, True, False)
        process(1, 1, True, False)

        def pair_body(i, _):
            process(2 * i, 0, True, True)
            process(2 * i + 1, 1, True, True)
            return 0
        lax.fori_loop(1, nfull // 2 - 1, pair_body, 0)

        process(nfull - 2, 0, True, True)
        process(nfull - 1, 1, False, True)

        if rem:
            rbase = wbase + nfull * CHUNK
            pltpu.make_async_copy(
                p_hbm.at[t_v.at[pl.ds(nfull * CHUNK, rem)]],
                rows_v.at[0, pl.ds(0, rem)], gsem[0]).start()
            pltpu.sync_copy(x_hbm.at[pl.ds(rbase, rem)],
                            x_v.at[0, pl.ds(0, rem)])
            pltpu.make_async_copy(
                p_hbm.at[t_v.at[pl.ds(nfull * CHUNK, rem)]],
                rows_v.at[0, pl.ds(0, rem)], gsem[0]).wait()

            def add_row_r(r, _):
                unpack_add_row(0, r)
                return 0
            lax.fori_loop(0, rem, add_row_r, 0)
            pltpu.sync_copy(o_v.at[0, pl.ds(0, rem)],
                            out_hbm.at[pl.ds(rbase, rem)])

        o_desc(nfull - 2, 0).wait()
        o_desc(nfull - 1, 1).wait()

    return k(P, t, x)


def kernel(x, t, emb, W, b):
    t = t.astype(jnp.int32)
    b2 = jnp.broadcast_to(b[None, :], (8, N_HID))
    P = _project_table(emb, W, b2)
    return _gather_add(P, t, x)


# speedup vs baseline: 1.3244x; 1.3244x over previous
import functools

import jax
import jax.numpy as jnp
from jax import lax
from jax.experimental import pallas as pl
from jax.experimental.pallas import tpu as pltpu
from jax.experimental.pallas import tpu_sc as plsc

N_HID = 128
LANES = 16
NC = 2
NS = 16
NW = NC * NS
CHUNK = 128


def _project_table(emb, W, b2):
    M = emb.shape[0]
    BM = 2048
    grid = (pl.cdiv(M, BM),)
    H = N_HID // 2

    def body(e_ref, w_ref, b_ref, o_ref):
        acc = jax.lax.dot_general(
            e_ref[...], w_ref[...],
            dimension_numbers=(((1,), (1,)), ((), ())),
            preferred_element_type=jnp.float32,
        )
        o_ref[...] = acc + b_ref[0, :][None, :]

    return pl.pallas_call(
        body,
        grid=grid,
        in_specs=[
            pl.BlockSpec((BM, N_HID), lambda i: (i, 0)),
            pl.BlockSpec((N_HID, N_HID), lambda i: (0, 0)),
            pl.BlockSpec((8, N_HID), lambda i: (0, 0)),
        ],
        out_specs=pl.BlockSpec((BM, N_HID), lambda i: (i, 0)),
        out_shape=jax.ShapeDtypeStruct((M, N_HID), jnp.float32),
    )(emb, W, b2)


def _gather_add(P, t, x):
    N = t.shape[0]
    assert N % NW == 0
    rows_w = N // NW
    nfull = rows_w // CHUNK
    rem = rows_w - nfull * CHUNK
    assert rem % 8 == 0 and nfull >= 4 and nfull % 2 == 0

    mesh = plsc.VectorSubcoreMesh(core_axis_name="c", subcore_axis_name="s")

    @functools.partial(
        pl.kernel,
        mesh=mesh,
        out_type=jax.ShapeDtypeStruct((N, N_HID), jnp.float32),
        compiler_params=pltpu.CompilerParams(
            needs_layout_passes=False, use_tc_tiling_on_sc=False),
        scratch_types=[
            pltpu.VMEM((rows_w,), jnp.int32),
            pltpu.VMEM((2, CHUNK, N_HID), jnp.float32),
            pltpu.VMEM((2, CHUNK, N_HID), jnp.float32),
            pltpu.VMEM((2, CHUNK, N_HID), jnp.float32),
            pltpu.SemaphoreType.DMA,
            pltpu.SemaphoreType.DMA,
            pltpu.SemaphoreType.DMA,
            pltpu.SemaphoreType.DMA,
            pltpu.SemaphoreType.DMA,
            pltpu.SemaphoreType.DMA,
        ],
    )
    def k(p_hbm, t_hbm, x_hbm, out_hbm, t_v, rows_v, x_v, o_v,
          g0, g1, xs0, xs1, os0, os1):
        wid = lax.axis_index("s") * NC + lax.axis_index("c")
        wbase = wid * rows_w
        gsem = (g0, g1)
        xsem = (xs0, xs1)
        osem = (os0, os1)

        pltpu.sync_copy(t_hbm.at[pl.ds(wbase, rows_w)], t_v)

        def gather_desc(c, s):
            return pltpu.make_async_copy(
                p_hbm.at[t_v.at[pl.ds(c * CHUNK, CHUNK)]], rows_v.at[s],
                gsem[s])

        def x_desc(c, s):
            return pltpu.make_async_copy(
                x_hbm.at[pl.ds(wbase + c * CHUNK, CHUNK)], x_v.at[s], xsem[s])

        def o_desc(c, s):
            return pltpu.make_async_copy(
                o_v.at[s], out_hbm.at[pl.ds(wbase + c * CHUNK, CHUNK)],
                osem[s])

        def issue(c, s):
            gather_desc(c, s).start()
            x_desc(c, s).start()

        def unpack_add_row(s, r):
            for j in range(N_HID // LANES):
                sl = pl.ds(j * LANES, LANES)
                o_v[s, r, sl] = x_v[s, r, sl] + rows_v[s, r, sl]

        def add_chunk(s):
            def add_row(r, _):
                unpack_add_row(s, r)
                return 0
            lax.fori_loop(0, CHUNK, add_row, 0)

        def process(c, s, prefetch, wait_store):
            gather_desc(c, s).wait()
            x_desc(c, s).wait()
            if prefetch:
                issue(c + 1, 1 - s)
            if wait_store:
                o_desc(c - 2, s).wait()
            add_chunk(s)
            o_desc(c, s).start()

        issue(0, 0)
        process(0, 0, True, False)
        process(1, 1, True, False)

        def pair_body(i, _):
            process(2 * i, 0, True, True)
            process(2 * i + 1, 1, True, True)
            return 0
        lax.fori_loop(1, nfull // 2 - 1, pair_body, 0)

        process(nfull - 2, 0, True, True)
        process(nfull - 1, 1, False, True)

        if rem:
            rbase = wbase + nfull * CHUNK
            pltpu.make_async_copy(
                p_hbm.at[t_v.at[pl.ds(nfull * CHUNK, rem)]],
                rows_v.at[0, pl.ds(0, rem)], gsem[0]).start()
            pltpu.sync_copy(x_hbm.at[pl.ds(rbase, rem)],
                            x_v.at[0, pl.ds(0, rem)])
            pltpu.make_async_copy(
                p_hbm.at[t_v.at[pl.ds(nfull * CHUNK, rem)]],
                rows_v.at[0, pl.ds(0, rem)], gsem[0]).wait()

            def add_row_r(r, _):
                unpack_add_row(0, r)
                return 0
            lax.fori_loop(0, rem, add_row_r, 0)
            pltpu.sync_copy(o_v.at[0, pl.ds(0, rem)],
                            out_hbm.at[pl.ds(rbase, rem)])

        o_desc(nfull - 2, 0).wait()
        o_desc(nfull - 1, 1).wait()

    return k(P, t, x)


def kernel(x, t, emb, W, b):
    t = t.astype(jnp.int32)
    b2 = jnp.broadcast_to(b[None, :], (8, N_HID))
    P = _project_table(emb, W, b2)
    return _gather_add(P, t, x)
